# baseline (speedup 1.0000x reference)
import jax
import jax.numpy as jnp
from jax.experimental import pallas as pl
from jax.experimental.pallas import tpu as pltpu

NUM_CLASSES = 256
FEAT = 512
SUP_BLOCK = 2048
Q_BLOCK = 4096


def _proto_kernel(labels_ref, sup_ref, out_ref, acc_ref, cnt_ref):
    i = pl.program_id(0)
    nsteps = pl.num_programs(0)
    labels = labels_ref[i]
    classes = jax.lax.broadcasted_iota(jnp.int32, (NUM_CLASSES, SUP_BLOCK), 0)
    onehot = (classes == labels[None, :]).astype(jnp.float32)
    sb = sup_ref[...].astype(jnp.bfloat16)
    partial = jax.lax.dot_general(
        onehot.astype(jnp.bfloat16), sb,
        dimension_numbers=(((1,), (0,)), ((), ())),
        preferred_element_type=jnp.float32)
    pcnt = jnp.sum(onehot, axis=1, keepdims=True)

    @pl.when(i == 0)
    def _init():
        acc_ref[...] = partial
        cnt_ref[...] = pcnt

    @pl.when(i > 0)
    def _acc():
        acc_ref[...] += partial
        cnt_ref[...] += pcnt

    @pl.when(i == nsteps - 1)
    def _finalize():
        protos = acc_ref[...] / jnp.maximum(cnt_ref[...], 1.0)
        out_ref[...] = protos.T


def _dist_kernel(q_ref, pt_ref, out_ref):
    pt = pt_ref[...]
    p2 = jnp.sum(pt * pt, axis=0, keepdims=True)
    qb = q_ref[...]
    q2 = jnp.sum(qb * qb, axis=1, keepdims=True)
    cross = jax.lax.dot_general(
        qb.astype(jnp.bfloat16), pt.astype(jnp.bfloat16),
        dimension_numbers=(((1,), (0,)), ((), ())),
        preferred_element_type=jnp.float32)
    d2 = (q2 + p2) - 2.0 * cross
    out_ref[...] = -jnp.sqrt(jnp.maximum(d2, 0.0))


@jax.jit
def kernel(support_features, support_labels, query_features):
    n_sup = support_features.shape[0]
    n_q = query_features.shape[0]
    labels2d = support_labels.astype(jnp.int32).reshape(
        n_sup // SUP_BLOCK, SUP_BLOCK)

    protoT = pl.pallas_call(
        _proto_kernel,
        grid=(n_sup // SUP_BLOCK,),
        in_specs=[
            pl.BlockSpec(labels2d.shape, lambda i: (0, 0)),
            pl.BlockSpec((SUP_BLOCK, FEAT), lambda i: (i, 0)),
        ],
        out_specs=pl.BlockSpec((FEAT, NUM_CLASSES), lambda i: (0, 0)),
        out_shape=jax.ShapeDtypeStruct((FEAT, NUM_CLASSES), jnp.float32),
        scratch_shapes=[
            pltpu.VMEM((NUM_CLASSES, FEAT), jnp.float32),
            pltpu.VMEM((NUM_CLASSES, 1), jnp.float32),
        ],
    )(labels2d, support_features)

    out = pl.pallas_call(
        _dist_kernel,
        grid=(n_q // Q_BLOCK,),
        in_specs=[
            pl.BlockSpec((Q_BLOCK, FEAT), lambda i: (i, 0)),
            pl.BlockSpec((FEAT, NUM_CLASSES), lambda i: (0, 0)),
        ],
        out_specs=pl.BlockSpec((Q_BLOCK, NUM_CLASSES), lambda i: (i, 0)),
        out_shape=jax.ShapeDtypeStruct((n_q, NUM_CLASSES), jnp.float32),
    )(query_features, protoT)
    return out


# speedup vs baseline: 1.0439x; 1.0439x over previous
import jax
import jax.numpy as jnp
from jax.experimental import pallas as pl
from jax.experimental.pallas import tpu as pltpu

NUM_CLASSES = 256
FEAT = 512
SUP_BLOCK = 4096
Q_BLOCK = 4096


def _proto_kernel(labels_ref, sup_ref, out_ref, acc_ref, cnt_ref):
    i = pl.program_id(0)
    nsteps = pl.num_programs(0)
    labels = labels_ref[i]
    classes = jax.lax.broadcasted_iota(jnp.int32, (NUM_CLASSES, SUP_BLOCK), 0)
    onehot = (classes == labels[None, :]).astype(jnp.float32)
    sb = sup_ref[...].astype(jnp.bfloat16)
    partial = jax.lax.dot_general(
        onehot.astype(jnp.bfloat16), sb,
        dimension_numbers=(((1,), (0,)), ((), ())),
        preferred_element_type=jnp.float32)
    pcnt = jnp.sum(onehot, axis=1, keepdims=True)

    @pl.when(i == 0)
    def _init():
        acc_ref[...] = partial
        cnt_ref[...] = pcnt

    @pl.when(i > 0)
    def _acc():
        acc_ref[...] += partial
        cnt_ref[...] += pcnt

    @pl.when(i == nsteps - 1)
    def _finalize():
        protos = acc_ref[...] / jnp.maximum(cnt_ref[...], 1.0)
        out_ref[...] = protos.T


def _dist_kernel(q_ref, pt_ref, out_ref):
    pt = pt_ref[...]
    p2 = jnp.sum(pt * pt, axis=0, keepdims=True)
    qb = q_ref[...]
    q2 = jnp.sum(qb * qb, axis=1, keepdims=True)
    cross = jax.lax.dot_general(
        qb.astype(jnp.bfloat16), pt.astype(jnp.bfloat16),
        dimension_numbers=(((1,), (0,)), ((), ())),
        preferred_element_type=jnp.float32)
    d2 = (q2 + p2) - 2.0 * cross
    out_ref[...] = -jnp.sqrt(jnp.maximum(d2, 0.0))


@jax.jit
def kernel(support_features, support_labels, query_features):
    n_sup = support_features.shape[0]
    n_q = query_features.shape[0]
    labels2d = support_labels.astype(jnp.int32).reshape(
        n_sup // SUP_BLOCK, SUP_BLOCK)

    protoT = pl.pallas_call(
        _proto_kernel,
        grid=(n_sup // SUP_BLOCK,),
        in_specs=[
            pl.BlockSpec(labels2d.shape, lambda i: (0, 0)),
            pl.BlockSpec((SUP_BLOCK, FEAT), lambda i: (i, 0)),
        ],
        out_specs=pl.BlockSpec((FEAT, NUM_CLASSES), lambda i: (0, 0)),
        out_shape=jax.ShapeDtypeStruct((FEAT, NUM_CLASSES), jnp.float32),
        scratch_shapes=[
            pltpu.VMEM((NUM_CLASSES, FEAT), jnp.float32),
            pltpu.VMEM((NUM_CLASSES, 1), jnp.float32),
        ],
    )(labels2d, support_features)

    out = pl.pallas_call(
        _dist_kernel,
        grid=(n_q // Q_BLOCK,),
        in_specs=[
            pl.BlockSpec((Q_BLOCK, FEAT), lambda i: (i, 0)),
            pl.BlockSpec((FEAT, NUM_CLASSES), lambda i: (0, 0)),
        ],
        out_specs=pl.BlockSpec((Q_BLOCK, NUM_CLASSES), lambda i: (i, 0)),
        out_shape=jax.ShapeDtypeStruct((n_q, NUM_CLASSES), jnp.float32),
    )(query_features, protoT)
    return out
